# Initial kernel scaffold; baseline (speedup 1.0000x reference)
import jax
import jax.numpy as jnp
from jax.experimental import pallas as pl


def kernel(data, labels):
    raise NotImplementedError("write your pallas kernel here")


# speedup vs baseline: 43.4898x; 43.4898x over previous
import functools

import jax
import jax.numpy as jnp
from jax import lax
from jax.experimental import pallas as pl

_DELTA_VAR = 0.5
_DELTA_DIST = 1.5
_VAR_W = 1.0
_DIST_W = 1.0
_REG_W = 0.001
_K = 16


def _pass_a_body(x_ref, lab_ref, sums_ref, cnts_ref):
    i = pl.program_id(0)
    x = x_ref[...]
    lab = lab_ref[0]
    kio = lax.broadcasted_iota(jnp.int32, (_K, x.shape[1]), 0)
    oh = (kio == lab).astype(jnp.float32)
    s = lax.dot_general(oh, x, (((1,), (1,)), ((), ())),
                        preferred_element_type=jnp.float32)
    c = jnp.broadcast_to(jnp.sum(oh, axis=1, keepdims=True), s.shape)

    @pl.when(i == 0)
    def _():
        sums_ref[...] = s
        cnts_ref[...] = c

    @pl.when(i > 0)
    def _():
        sums_ref[...] += s
        cnts_ref[...] += c


def _pass_b_body(x_ref, lab_ref, sums_ref, cnts_ref, loss_ref):
    i = pl.program_id(0)
    nb = pl.num_programs(0)
    centers = sums_ref[...] / cnts_ref[...]
    x = x_ref[...]
    lab = lab_ref[0]
    c_sz = x.shape[1]

    cn2 = jnp.sum(centers * centers, axis=1, keepdims=True)
    p2 = lax.dot_general(2.0 * centers, x, (((1,), (0,)), ((), ())),
                         preferred_element_type=jnp.float32)
    w = p2 - cn2
    kio = lax.broadcasted_iota(jnp.int32, (_K, c_sz), 0)
    sel = jnp.sum(jnp.where(kio == lab, w, 0.0), axis=0, keepdims=True)
    q = jnp.sum(x * x, axis=0, keepdims=True)
    d2 = jnp.maximum(q - sel, 0.0)
    dist = jnp.sqrt(d2 + 1e-12)
    hinge = jnp.maximum(dist - _DELTA_VAR, 0.0)
    chunk = jnp.sum(hinge * hinge).reshape(1, 1)

    @pl.when(i == 0)
    def _():
        loss_ref[...] = jnp.zeros((1, 1), jnp.float32)

    loss_ref[...] += chunk

    @pl.when(i == nb - 1)
    def _():
        var_term = loss_ref[...] / _K
        sum_dist = 0.0
        kio1 = lax.broadcasted_iota(jnp.int32, (_K, 1), 0)
        for k in range(_K):
            dk = centers - centers[k:k + 1, :]
            sqk = jnp.sum(dk * dk, axis=1, keepdims=True)
            eyek = (kio1 == k).astype(jnp.float32)
            cd = jnp.sqrt(sqk + eyek)
            t = jnp.maximum(2.0 * _DELTA_DIST - cd, 0.0)
            sum_dist += jnp.sum(t * t * (1.0 - eyek))
        dist_term = sum_dist / (_K * (_K - 1))
        reg_term = jnp.sum(jnp.sqrt(cn2 + 1e-12)) / _K
        loss_ref[...] = (_VAR_W * var_term
                         + (_DIST_W * dist_term
                            + _REG_W * reg_term).reshape(1, 1))


@jax.jit
def kernel(data, labels):
    d = data.shape[0]
    n = data.shape[1] * data.shape[2]
    x = data.reshape(d, n)
    c_sz = min(16384, n)
    nb = n // c_sz
    lab3 = labels.reshape(nb, 1, c_sz)

    sums, cnts = pl.pallas_call(
        _pass_a_body,
        grid=(nb,),
        in_specs=[
            pl.BlockSpec((d, c_sz), lambda i: (0, i)),
            pl.BlockSpec((1, 1, c_sz), lambda i: (i, 0, 0)),
        ],
        out_specs=[
            pl.BlockSpec((_K, d), lambda i: (0, 0)),
            pl.BlockSpec((_K, d), lambda i: (0, 0)),
        ],
        out_shape=[
            jax.ShapeDtypeStruct((_K, d), jnp.float32),
            jax.ShapeDtypeStruct((_K, d), jnp.float32),
        ],
    )(x, lab3)

    loss = pl.pallas_call(
        _pass_b_body,
        grid=(nb,),
        in_specs=[
            pl.BlockSpec((d, c_sz), lambda i: (0, i)),
            pl.BlockSpec((1, 1, c_sz), lambda i: (i, 0, 0)),
            pl.BlockSpec((_K, d), lambda i: (0, 0)),
            pl.BlockSpec((_K, d), lambda i: (0, 0)),
        ],
        out_specs=pl.BlockSpec((1, 1), lambda i: (0, 0)),
        out_shape=jax.ShapeDtypeStruct((1, 1), jnp.float32),
    )(x, lab3, sums, cnts)

    return loss[0, 0]


# speedup vs baseline: 59.7725x; 1.3744x over previous
import functools

import jax
import jax.numpy as jnp
from jax import lax
from jax.experimental import pallas as pl

_DELTA_VAR = 0.5
_DELTA_DIST = 1.5
_VAR_W = 1.0
_DIST_W = 1.0
_REG_W = 0.001
_K = 16


def _pass_a_body(x_ref, lab_ref, sums_ref, cnts_ref):
    i = pl.program_id(0)
    x3 = x_ref[...]
    x = x3.reshape(x3.shape[0], x3.shape[1] * x3.shape[2])
    lab2 = lab_ref[...]
    lab = lab2.reshape(1, lab2.shape[0] * lab2.shape[1])
    kio = lax.broadcasted_iota(jnp.int32, (_K, x.shape[1]), 0)
    oh = (kio == lab).astype(jnp.float32)
    s = lax.dot_general(oh, x, (((1,), (1,)), ((), ())),
                        preferred_element_type=jnp.float32)
    c = jnp.broadcast_to(jnp.sum(oh, axis=1, keepdims=True), s.shape)

    @pl.when(i == 0)
    def _():
        sums_ref[...] = s
        cnts_ref[...] = c

    @pl.when(i > 0)
    def _():
        sums_ref[...] += s
        cnts_ref[...] += c


def _pass_b_body(x_ref, lab_ref, sums_ref, cnts_ref, loss_ref):
    i = pl.program_id(0)
    nb = pl.num_programs(0)
    centers = sums_ref[...] / cnts_ref[...]
    x3 = x_ref[...]
    x = x3.reshape(x3.shape[0], x3.shape[1] * x3.shape[2])
    lab2 = lab_ref[...]
    lab = lab2.reshape(1, lab2.shape[0] * lab2.shape[1])
    c_sz = x.shape[1]

    cn2 = jnp.sum(centers * centers, axis=1, keepdims=True)
    p2 = lax.dot_general(2.0 * centers, x, (((1,), (0,)), ((), ())),
                         preferred_element_type=jnp.float32)
    w = p2 - cn2
    kio = lax.broadcasted_iota(jnp.int32, (_K, c_sz), 0)
    sel = jnp.sum(jnp.where(kio == lab, w, 0.0), axis=0, keepdims=True)
    q = jnp.sum(x * x, axis=0, keepdims=True)
    d2 = jnp.maximum(q - sel, 0.0)
    dist = jnp.sqrt(d2 + 1e-12)
    hinge = jnp.maximum(dist - _DELTA_VAR, 0.0)
    chunk = jnp.sum(hinge * hinge).reshape(1, 1)

    @pl.when(i == 0)
    def _():
        loss_ref[...] = jnp.zeros((1, 1), jnp.float32)

    loss_ref[...] += chunk

    @pl.when(i == nb - 1)
    def _():
        var_term = loss_ref[...] / _K
        sum_dist = 0.0
        kio1 = lax.broadcasted_iota(jnp.int32, (_K, 1), 0)
        for k in range(_K):
            dk = centers - centers[k:k + 1, :]
            sqk = jnp.sum(dk * dk, axis=1, keepdims=True)
            eyek = (kio1 == k).astype(jnp.float32)
            cd = jnp.sqrt(sqk + eyek)
            t = jnp.maximum(2.0 * _DELTA_DIST - cd, 0.0)
            sum_dist += jnp.sum(t * t * (1.0 - eyek))
        dist_term = sum_dist / (_K * (_K - 1))
        reg_term = jnp.sum(jnp.sqrt(cn2 + 1e-12)) / _K
        loss_ref[...] = (_VAR_W * var_term
                         + (_DIST_W * dist_term
                            + _REG_W * reg_term).reshape(1, 1))


@jax.jit
def kernel(data, labels):
    d, h, w = data.shape
    hb = min(16, h)
    nb = h // hb

    sums, cnts = pl.pallas_call(
        _pass_a_body,
        grid=(nb,),
        in_specs=[
            pl.BlockSpec((d, hb, w), lambda i: (0, i, 0)),
            pl.BlockSpec((hb, w), lambda i: (i, 0)),
        ],
        out_specs=[
            pl.BlockSpec((_K, d), lambda i: (0, 0)),
            pl.BlockSpec((_K, d), lambda i: (0, 0)),
        ],
        out_shape=[
            jax.ShapeDtypeStruct((_K, d), jnp.float32),
            jax.ShapeDtypeStruct((_K, d), jnp.float32),
        ],
    )(data, labels)

    loss = pl.pallas_call(
        _pass_b_body,
        grid=(nb,),
        in_specs=[
            pl.BlockSpec((d, hb, w), lambda i: (0, i, 0)),
            pl.BlockSpec((hb, w), lambda i: (i, 0)),
            pl.BlockSpec((_K, d), lambda i: (0, 0)),
            pl.BlockSpec((_K, d), lambda i: (0, 0)),
        ],
        out_specs=pl.BlockSpec((1, 1), lambda i: (0, 0)),
        out_shape=jax.ShapeDtypeStruct((1, 1), jnp.float32),
    )(data, labels, sums, cnts)

    return loss[0, 0]
